# baseline (speedup 1.0000x reference)
import functools

import jax
import jax.numpy as jnp
from jax import lax
from jax.experimental import pallas as pl
from jax.experimental.pallas import tpu as pltpu
from jax.experimental.pallas import tpu_sc as plsc

SEQ_LEN = 4096
BATCH = 4
HIDDEN = 1024
NUM_CORES = 2
NUM_SUBCORES = 16
NUM_WORKERS = NUM_CORES * NUM_SUBCORES
ROWS_TOTAL = SEQ_LEN * BATCH
ROWS_PER_WORKER = ROWS_TOTAL // NUM_WORKERS
CHUNK = 32
NCHUNKS = ROWS_PER_WORKER // CHUNK

_mesh = plsc.VectorSubcoreMesh(
    core_axis_name="c", subcore_axis_name="s",
    num_cores=NUM_CORES, num_subcores=NUM_SUBCORES,
)


@functools.partial(
    pl.kernel,
    out_type=jax.ShapeDtypeStruct(
        (NUM_WORKERS, NCHUNKS, CHUNK, HIDDEN), jnp.float32),
    mesh=_mesh,
    scratch_types=[
        pltpu.VMEM((NCHUNKS, CHUNK), jnp.int32),
        pltpu.VMEM((CHUNK, HIDDEN), jnp.float32),
        pltpu.VMEM((CHUNK, HIDDEN), jnp.float32),
        pltpu.SemaphoreType.DMA,
        pltpu.SemaphoreType.DMA,
        pltpu.SemaphoreType.DMA,
        pltpu.SemaphoreType.DMA,
    ],
)
def _sc_lookup(idx_hbm, table_hbm, out_hbm, idx_v, buf0, buf1,
               gs0, gs1, ws0, ws1):
    wid = lax.axis_index("s") * NUM_CORES + lax.axis_index("c")
    pltpu.sync_copy(idx_hbm.at[wid], idx_v)

    bufs = (buf0, buf1)
    gsems = (gs0, gs1)
    wsems = (ws0, ws1)

    gh = [None] * NCHUNKS
    wh = [None] * NCHUNKS
    gh[0] = pltpu.async_copy(table_hbm.at[idx_v.at[0]], bufs[0], gsems[0])
    gh[1] = pltpu.async_copy(table_hbm.at[idx_v.at[1]], bufs[1], gsems[1])
    for c in range(NCHUNKS):
        b = c % 2
        gh[c].wait()
        wh[c] = pltpu.async_copy(bufs[b], out_hbm.at[wid, c], wsems[b])
        if c + 2 < NCHUNKS:
            wh[c].wait()
            gh[c + 2] = pltpu.async_copy(
                table_hbm.at[idx_v.at[c + 2]], bufs[b], gsems[b])
    wh[NCHUNKS - 2].wait()
    wh[NCHUNKS - 1].wait()


def kernel(token_type_ids, segment_embedding_weight):
    ids = token_type_ids.reshape(NUM_WORKERS, NCHUNKS, CHUNK).astype(jnp.int32)
    out = _sc_lookup(ids, segment_embedding_weight)
    return out.reshape(SEQ_LEN, BATCH, HIDDEN)


# speedup vs baseline: 3.6564x; 3.6564x over previous
import functools

import jax
import jax.numpy as jnp
from jax import lax
from jax.experimental import pallas as pl
from jax.experimental.pallas import tpu as pltpu
from jax.experimental.pallas import tpu_sc as plsc

SEQ_LEN = 4096
BATCH = 4
HIDDEN = 1024
LANES = 16
JCHUNKS = HIDDEN // LANES
NUM_CORES = 2
NUM_SUBCORES = 16
NUM_WORKERS = NUM_CORES * NUM_SUBCORES
ROWS_TOTAL = SEQ_LEN * BATCH
ROWS_PER_WORKER = ROWS_TOTAL // NUM_WORKERS
CHUNK = 32
NCHUNKS = ROWS_PER_WORKER // CHUNK
GROUP = 16
NGROUPS = CHUNK // GROUP

_mesh = plsc.VectorSubcoreMesh(
    core_axis_name="c", subcore_axis_name="s",
    num_cores=NUM_CORES, num_subcores=NUM_SUBCORES,
)


@functools.partial(
    pl.kernel,
    out_type=jax.ShapeDtypeStruct(
        (NUM_WORKERS, NCHUNKS, CHUNK * HIDDEN), jnp.float32),
    mesh=_mesh,
    scratch_types=[
        pltpu.VMEM((ROWS_PER_WORKER,), jnp.int32),
        pltpu.VMEM((2 * HIDDEN,), jnp.float32),
        pltpu.VMEM((CHUNK * HIDDEN,), jnp.float32),
        pltpu.VMEM((CHUNK * HIDDEN,), jnp.float32),
        pltpu.SemaphoreType.DMA,
        pltpu.SemaphoreType.DMA,
    ],
)
def _sc_lookup(idx_hbm, table_hbm, out_hbm, idx_v, table_v, buf0, buf1,
               ws0, ws1):
    wid = lax.axis_index("s") * NUM_CORES + lax.axis_index("c")
    pltpu.sync_copy(idx_hbm.at[pl.ds(wid * ROWS_PER_WORKER, ROWS_PER_WORKER)],
                    idx_v)
    pltpu.sync_copy(table_hbm, table_v)

    bufs = (buf0, buf1)
    wsems = (ws0, ws1)

    def build_chunk(c, buf):
        for g in range(NGROUPS):
            row0 = c * CHUNK + g * GROUP
            ids_vec = idx_v[pl.ds(row0, GROUP)]
            masks = []
            for r in range(GROUP):
                masks.append(ids_vec[r] == 1)
            base = g * GROUP * HIDDEN

            @pl.loop(0, JCHUNKS)
            def _(j):
                off = j * LANES
                w0 = table_v[pl.ds(off, LANES)]
                w1 = table_v[pl.ds(HIDDEN + off, LANES)]
                for r in range(GROUP):
                    buf[pl.ds(base + r * HIDDEN + off, LANES)] = (
                        jnp.where(masks[r], w1, w0))

    @pl.loop(0, NCHUNKS, step=2)
    def _(c0):
        for p in range(2):
            c = c0 + p

            @pl.when(c >= 2)
            def _():
                pltpu.make_async_copy(
                    bufs[p], out_hbm.at[wid, c - 2], wsems[p]).wait()

            build_chunk(c, bufs[p])
            pltpu.async_copy(bufs[p], out_hbm.at[wid, c], wsems[p])

    pltpu.make_async_copy(
        bufs[0], out_hbm.at[wid, NCHUNKS - 2], wsems[0]).wait()
    pltpu.make_async_copy(
        bufs[1], out_hbm.at[wid, NCHUNKS - 1], wsems[1]).wait()


def kernel(token_type_ids, segment_embedding_weight):
    ids = token_type_ids.reshape(ROWS_TOTAL).astype(jnp.int32)
    table = segment_embedding_weight.reshape(2 * HIDDEN)
    out = _sc_lookup(ids, table)
    return out.reshape(SEQ_LEN, BATCH, HIDDEN)


# speedup vs baseline: 8.8496x; 2.4203x over previous
import functools

import jax
import jax.numpy as jnp
from jax import lax
from jax.experimental import pallas as pl
from jax.experimental.pallas import tpu as pltpu
from jax.experimental.pallas import tpu_sc as plsc

SEQ_LEN = 4096
BATCH = 4
HIDDEN = 1024
LANES = 16
JCHUNKS = HIDDEN // LANES
NUM_CORES = 2
NUM_SUBCORES = 16
NUM_WORKERS = NUM_CORES * NUM_SUBCORES
ROWS_TOTAL = SEQ_LEN * BATCH
ROWS_PER_WORKER = ROWS_TOTAL // NUM_WORKERS
CHUNK = 32
SEQ_PER_CHUNK = CHUNK // BATCH
NCHUNKS = ROWS_PER_WORKER // CHUNK
GROUP = 16
NGROUPS = CHUNK // GROUP

_mesh = plsc.VectorSubcoreMesh(
    core_axis_name="c", subcore_axis_name="s",
    num_cores=NUM_CORES, num_subcores=NUM_SUBCORES,
)


@functools.partial(
    pl.kernel,
    out_type=jax.ShapeDtypeStruct((SEQ_LEN, BATCH, HIDDEN), jnp.float32),
    mesh=_mesh,
    scratch_types=[
        pltpu.VMEM((ROWS_PER_WORKER,), jnp.int32),
        pltpu.VMEM((2 * HIDDEN,), jnp.float32),
        pltpu.VMEM((SEQ_PER_CHUNK, BATCH, HIDDEN), jnp.float32),
        pltpu.VMEM((SEQ_PER_CHUNK, BATCH, HIDDEN), jnp.float32),
        pltpu.SemaphoreType.DMA,
        pltpu.SemaphoreType.DMA,
    ],
)
def _sc_lookup(idx_hbm, table_hbm, out_hbm, idx_v, table_v, buf0, buf1,
               ws0, ws1):
    wid = lax.axis_index("s") * NUM_CORES + lax.axis_index("c")
    pltpu.sync_copy(idx_hbm.at[pl.ds(wid * ROWS_PER_WORKER, ROWS_PER_WORKER)],
                    idx_v)
    pltpu.sync_copy(table_hbm, table_v)

    bufs = (buf0, buf1)
    wsems = (ws0, ws1)
    seq0 = wid * (ROWS_PER_WORKER // BATCH)

    def build_chunk(c, buf):
        for g in range(NGROUPS):
            row0 = c * CHUNK + g * GROUP
            ids_vec = idx_v[pl.ds(row0, GROUP)]
            masks = []
            for r in range(GROUP):
                masks.append(ids_vec[r] == 1)

            @pl.loop(0, JCHUNKS)
            def _(j):
                off = j * LANES
                w0 = table_v[pl.ds(off, LANES)]
                w1 = table_v[pl.ds(HIDDEN + off, LANES)]
                for r in range(GROUP):
                    flat = g * GROUP + r
                    buf[flat // BATCH, flat % BATCH, pl.ds(off, LANES)] = (
                        jnp.where(masks[r], w1, w0))

    @pl.loop(0, NCHUNKS, step=2)
    def _(c0):
        for p in range(2):
            c = c0 + p
            dst = out_hbm.at[pl.ds(seq0 + c * SEQ_PER_CHUNK, SEQ_PER_CHUNK)]

            @pl.when(c >= 2)
            def _():
                pltpu.make_async_copy(
                    bufs[p],
                    out_hbm.at[
                        pl.ds(seq0 + (c - 2) * SEQ_PER_CHUNK, SEQ_PER_CHUNK)],
                    wsems[p]).wait()

            build_chunk(c, bufs[p])
            pltpu.async_copy(bufs[p], dst, wsems[p])

    pltpu.make_async_copy(
        bufs[0],
        out_hbm.at[pl.ds(seq0 + (NCHUNKS - 2) * SEQ_PER_CHUNK, SEQ_PER_CHUNK)],
        wsems[0]).wait()
    pltpu.make_async_copy(
        bufs[1],
        out_hbm.at[pl.ds(seq0 + (NCHUNKS - 1) * SEQ_PER_CHUNK, SEQ_PER_CHUNK)],
        wsems[1]).wait()


def kernel(token_type_ids, segment_embedding_weight):
    ids = token_type_ids.reshape(ROWS_TOTAL).astype(jnp.int32)
    table = segment_embedding_weight.reshape(2 * HIDDEN)
    return _sc_lookup(ids, table)


# speedup vs baseline: 10.1361x; 1.1454x over previous
import functools

import jax
import jax.numpy as jnp
from jax import lax
from jax.experimental import pallas as pl
from jax.experimental.pallas import tpu as pltpu
from jax.experimental.pallas import tpu_sc as plsc

SEQ_LEN = 4096
BATCH = 4
HIDDEN = 1024
LANES = 16
JCHUNKS = HIDDEN // LANES
NUM_CORES = 2
NUM_SUBCORES = 16
NUM_WORKERS = NUM_CORES * NUM_SUBCORES
ROWS_TOTAL = SEQ_LEN * BATCH
ROWS_PER_WORKER = ROWS_TOTAL // NUM_WORKERS
CHUNK = 16
SEQ_PER_CHUNK = CHUNK // BATCH
NCHUNKS = ROWS_PER_WORKER // CHUNK
GROUP = 16
NGROUPS = CHUNK // GROUP
NBUF = 4

_mesh = plsc.VectorSubcoreMesh(
    core_axis_name="c", subcore_axis_name="s",
    num_cores=NUM_CORES, num_subcores=NUM_SUBCORES,
)


@functools.partial(
    pl.kernel,
    out_type=jax.ShapeDtypeStruct((SEQ_LEN, BATCH, HIDDEN), jnp.float32),
    mesh=_mesh,
    scratch_types=[
        pltpu.VMEM((ROWS_PER_WORKER,), jnp.int32),
        pltpu.VMEM((2, HIDDEN), jnp.float32),
        pltpu.VMEM((NBUF, SEQ_PER_CHUNK, BATCH, HIDDEN), jnp.float32),
        pltpu.SemaphoreType.DMA,
        pltpu.SemaphoreType.DMA,
        pltpu.SemaphoreType.DMA,
        pltpu.SemaphoreType.DMA,
    ],
)
def _sc_lookup(idx_hbm, table_hbm, out_hbm, idx_v, table_v, bufv,
               ws0, ws1, ws2, ws3):
    wid = lax.axis_index("s") * NUM_CORES + lax.axis_index("c")
    pltpu.sync_copy(idx_hbm.at[pl.ds(wid * ROWS_PER_WORKER, ROWS_PER_WORKER)],
                    idx_v)
    pltpu.sync_copy(table_hbm, table_v)

    wsems = (ws0, ws1, ws2, ws3)
    seq0 = wid * (ROWS_PER_WORKER // BATCH)

    def build_chunk(c, buf):
        for g in range(NGROUPS):
            row0 = c * CHUNK + g * GROUP
            ids_vec = idx_v[pl.ds(row0, GROUP)]
            masks = []
            for r in range(GROUP):
                masks.append(ids_vec[r] == 1)

            @pl.loop(0, JCHUNKS)
            def _(j):
                off = j * LANES
                w0 = table_v[0, pl.ds(off, LANES)]
                w1 = table_v[1, pl.ds(off, LANES)]
                for r in range(GROUP):
                    flat = g * GROUP + r
                    buf[flat // BATCH, flat % BATCH, pl.ds(off, LANES)] = (
                        jnp.where(masks[r], w1, w0))

    def out_slice(c):
        return out_hbm.at[pl.ds(seq0 + c * SEQ_PER_CHUNK, SEQ_PER_CHUNK)]

    @pl.loop(0, NCHUNKS, step=NBUF)
    def _(c0):
        for p in range(NBUF):
            c = c0 + p

            @pl.when(c >= NBUF)
            def _():
                pltpu.make_async_copy(
                    bufv.at[p], out_slice(c - NBUF), wsems[p]).wait()

            build_chunk(c, bufv.at[p])
            pltpu.async_copy(bufv.at[p], out_slice(c), wsems[p])

    for p in range(NBUF):
        pltpu.make_async_copy(
            bufv.at[p], out_slice(NCHUNKS - NBUF + p), wsems[p]).wait()


def kernel(token_type_ids, segment_embedding_weight):
    ids = token_type_ids.reshape(ROWS_TOTAL).astype(jnp.int32)
    return _sc_lookup(ids, segment_embedding_weight)
